# Initial kernel scaffold; baseline (speedup 1.0000x reference)
import jax
import jax.numpy as jnp
from jax.experimental import pallas as pl


def kernel(node_embed, solution, costs, dist, solution_embed_old, Wq, Wk, mix1_weight, mix1_bias, mix2_weight, mix2_bias, norm_head_w, gru_w_ih, gru_w_hh, gru_b_ih, gru_b_hh):
    raise NotImplementedError("write your pallas kernel here")


# speedup vs baseline: 6.1449x; 6.1449x over previous
import functools
import jax
import jax.numpy as jnp
from jax import lax
from jax.experimental import pallas as pl


def _gat_body(ne_ref, dist_ref, old_ref, solA_ref, nxtA_ref, solT_ref,
              nxtT_ref, costs_ref, wqT_ref, wkT_ref, m1w0_ref, m1w1_ref,
              m1b_ref, m2w_ref, m2b_ref, nhw_ref, wihT_ref, whhT_ref,
              bih_ref, bhh_ref, o1_ref, o2_ref, *, ns, nh, kd):
    ne = ne_ref[0]
    dist = dist_ref[0]
    old = old_ref[0]
    gs, d = ne.shape

    q_all = jnp.dot(ne, wqT_ref[...], preferred_element_type=jnp.float32)
    k_all = jnp.dot(ne, wkT_ref[...], preferred_element_type=jnp.float32)

    col_j = lax.broadcasted_iota(jnp.int32, (gs, gs), 1)
    row_i = lax.broadcasted_iota(jnp.int32, (gs, gs), 0)

    att = jnp.zeros((gs, gs), jnp.float32)
    for i in range(ns):
        s_col = solT_ref[0][:, i:i + 1]
        n_col = nxtT_ref[0][:, i:i + 1]
        s_row = solA_ref[0][i:i + 1, :]
        n_row = nxtA_ref[0][i:i + 1, :]

        ps = (col_j == s_col).astype(jnp.float32)
        pn = (col_j == n_col).astype(jnp.float32)

        qg = jnp.dot(ps, q_all, preferred_element_type=jnp.float32)
        kg = jnp.dot(pn, k_all, preferred_element_type=jnp.float32)
        prod = qg * kg
        dph = prod.reshape(gs, nh, kd).sum(axis=-1) * (1.0 / kd)

        dg = jnp.dot(ps, dist, preferred_element_type=jnp.float32)
        ec = jnp.sum(dg * pn, axis=1, keepdims=True)

        h1 = jnp.maximum(
            dph[:, :, None] * m1w0_ref[...][None]
            + ec[:, :, None] * m1w1_ref[...][None]
            + m1b_ref[...][None], 0.0)
        mixed = jnp.sum(h1 * m2w_ref[...][None], axis=2) + m2b_ref[...]
        w_t = jnp.sum(mixed * nhw_ref[...], axis=1, keepdims=True)
        cost_i = w_t / costs_ref[0][0:1, i:i + 1]

        key_col = s_col * gs + n_col
        key_row = s_row * gs + n_row
        dupe = (key_col == key_row) & (col_j > row_i)
        loser = jnp.sum(dupe.astype(jnp.float32), axis=1, keepdims=True) > 0.0
        val = jnp.where(loser, 0.0, cost_i)

        pst = (row_i == s_row).astype(jnp.float32)
        att = att + jnp.dot(pst, val * pn, preferred_element_type=jnp.float32)

    neg = jnp.float32(-9000000000000000.0)
    att = jnp.where(att != 0.0, att, neg)
    att = jnp.where(row_i == col_j, neg, att)
    m = jnp.max(att, axis=1, keepdims=True)
    e = jnp.exp(att - m)
    attn = e / jnp.sum(e, axis=1, keepdims=True)

    se = jnp.dot(attn, ne, preferred_element_type=jnp.float32)
    gi = jnp.dot(se, wihT_ref[...], preferred_element_type=jnp.float32) \
        + bih_ref[...]
    gh = jnp.dot(old, whhT_ref[...], preferred_element_type=jnp.float32) \
        + bhh_ref[...]
    r = jax.nn.sigmoid(gi[:, :d] + gh[:, :d])
    z = jax.nn.sigmoid(gi[:, d:2 * d] + gh[:, d:2 * d])
    n = jnp.tanh(gi[:, 2 * d:] + r * gh[:, 2 * d:])
    hnew = (1.0 - z) * n + z * old
    o2_ref[0] = hnew
    o1_ref[0] = jnp.where(hnew > 0.0, hnew, jnp.exp(hnew) - 1.0)


def kernel(node_embed, solution, costs, dist, solution_embed_old, Wq, Wk,
           mix1_weight, mix1_bias, mix2_weight, mix2_bias, norm_head_w,
           gru_w_ih, gru_w_hh, gru_b_ih, gru_b_hh, interpret=False):
    bs, gs, d = node_embed.shape
    ns = solution.shape[0]
    nh = mix1_weight.shape[0]
    kd = d // nh

    solution = solution.astype(jnp.int32)
    nxt = jnp.concatenate([solution[:, :, 1:], solution[:, :, 0:1]], axis=-1)
    sol_a = jnp.transpose(solution, (1, 0, 2))
    nxt_a = jnp.transpose(nxt, (1, 0, 2))
    sol_t = jnp.transpose(solution, (1, 2, 0))
    nxt_t = jnp.transpose(nxt, (1, 2, 0))
    costs3 = jnp.transpose(costs)[:, None, :]

    full = lambda *shape: pl.BlockSpec(shape, lambda b: (0,) * len(shape))
    grid_spec = pl.GridSpec(
        grid=(bs,),
        in_specs=[
            pl.BlockSpec((1, gs, d), lambda b: (b, 0, 0)),
            pl.BlockSpec((1, gs, gs), lambda b: (b, 0, 0)),
            pl.BlockSpec((1, gs, d), lambda b: (b, 0, 0)),
            pl.BlockSpec((1, ns, gs), lambda b: (b, 0, 0)),
            pl.BlockSpec((1, ns, gs), lambda b: (b, 0, 0)),
            pl.BlockSpec((1, gs, ns), lambda b: (b, 0, 0)),
            pl.BlockSpec((1, gs, ns), lambda b: (b, 0, 0)),
            pl.BlockSpec((1, 1, ns), lambda b: (b, 0, 0)),
            full(d, d), full(d, d),
            full(nh, 16), full(nh, 16), full(nh, 16),
            full(nh, 16), full(1, nh), full(1, nh),
            full(d, 3 * d), full(d, 3 * d),
            full(1, 3 * d), full(1, 3 * d),
        ],
        out_specs=[
            pl.BlockSpec((1, gs, d), lambda b: (b, 0, 0)),
            pl.BlockSpec((1, gs, d), lambda b: (b, 0, 0)),
        ],
    )
    out1, out2 = pl.pallas_call(
        functools.partial(_gat_body, ns=ns, nh=nh, kd=kd),
        grid_spec=grid_spec,
        out_shape=[
            jax.ShapeDtypeStruct((bs, gs, d), jnp.float32),
            jax.ShapeDtypeStruct((bs, gs, d), jnp.float32),
        ],
        interpret=interpret,
    )(node_embed, dist, solution_embed_old, sol_a, nxt_a, sol_t, nxt_t,
      costs3, Wq.T, Wk.T, mix1_weight[:, 0, :], mix1_weight[:, 1, :],
      mix1_bias, mix2_weight[:, :, 0], mix2_bias.T, norm_head_w,
      gru_w_ih.T, gru_w_hh.T, gru_b_ih[None, :], gru_b_hh[None, :])
    return (out1, out2)
